# Initial kernel scaffold; baseline (speedup 1.0000x reference)
import functools

import jax
import jax.numpy as jnp
from jax import lax
from jax.experimental import pallas as pl
from jax.experimental.pallas import tpu as pltpu
from jax.experimental.pallas import tpu_sc as plsc

SEQ_LEN = 512
CLS_ID = 101
SEP_ID = 102
PAD_ID = 0
B = 16
TOTAL = 4096
BUDGET = SEQ_LEN - 3
HALF = SEQ_LEN // 2
LANES = 16
STEPS = HALF // LANES

_mesh = plsc.VectorSubcoreMesh(core_axis_name="c", subcore_axis_name="s")


@functools.partial(
    pl.kernel,
    mesh=_mesh,
    out_type=[
        jax.ShapeDtypeStruct((B, SEQ_LEN), jnp.int32),
        jax.ShapeDtypeStruct((B, SEQ_LEN), jnp.int32),
        jax.ShapeDtypeStruct((B, SEQ_LEN), jnp.int32),
    ],
    scratch_types=[
        pltpu.VMEM((B + 1,), jnp.int32),
        pltpu.VMEM((B + 1,), jnp.int32),
        pltpu.VMEM((TOTAL,), jnp.int32),
        pltpu.VMEM((TOTAL,), jnp.int32),
        pltpu.VMEM((HALF,), jnp.int32),
        pltpu.VMEM((HALF,), jnp.int32),
        pltpu.VMEM((HALF,), jnp.int32),
    ],
)
def _pack_kernel(tok1_hbm, cu1_hbm, tok2_hbm, cu2_hbm,
                 ids_hbm, mask_hbm, type_hbm,
                 cu1_v, cu2_v, tok1_v, tok2_v, ids_v, mask_v, type_v):
    wid = lax.axis_index("s") * 2 + lax.axis_index("c")
    b = wid // 2
    h = wid % 2

    pltpu.sync_copy(cu1_hbm, cu1_v)
    pltpu.sync_copy(cu2_hbm, cu2_v)
    pltpu.sync_copy(tok1_hbm, tok1_v)
    pltpu.sync_copy(tok2_hbm, tok2_v)

    bv = jnp.full((LANES,), b, jnp.int32)
    c1lo = plsc.load_gather(cu1_v, [bv])
    c1hi = plsc.load_gather(cu1_v, [bv + 1])
    c2lo = plsc.load_gather(cu2_v, [bv])
    c2hi = plsc.load_gather(cu2_v, [bv + 1])
    len1 = c1hi - c1lo
    len2 = c2hi - c2lo
    t1 = jnp.minimum(len1, BUDGET - jnp.minimum(len2, BUDGET // 2))
    t2 = jnp.minimum(len2, BUDGET - t1)
    tsum2 = t1 + t2 + 2

    base = h * HALF
    lane = lax.iota(jnp.int32, LANES)
    for j in range(STEPS):
        p = lane + (j * LANES) + base
        in1 = (p >= 1) & (p <= t1)
        sep1 = p == t1 + 1
        in2 = (p >= t1 + 2) & (p <= tsum2 - 1)
        sep2 = p == tsum2
        idx1 = jnp.clip(c1lo + p - 1, 0, TOTAL - 1)
        idx2 = jnp.clip(c2lo + p - t1 - 2, 0, TOTAL - 1)
        g1 = plsc.load_gather(tok1_v, [idx1])
        g2 = plsc.load_gather(tok2_v, [idx2])
        ids = jnp.where(p == 0, CLS_ID,
              jnp.where(in1, g1,
              jnp.where(sep1, SEP_ID,
              jnp.where(in2, g2,
              jnp.where(sep2, SEP_ID, PAD_ID))))).astype(jnp.int32)
        mask = (p <= tsum2).astype(jnp.int32)
        tids = ((p >= t1 + 2) & (p <= tsum2)).astype(jnp.int32)
        sl = pl.ds(j * LANES, LANES)
        ids_v[sl] = ids
        mask_v[sl] = mask
        type_v[sl] = tids

    out_sl = pl.ds(base, HALF)
    pltpu.sync_copy(ids_v, ids_hbm.at[b, out_sl])
    pltpu.sync_copy(mask_v, mask_hbm.at[b, out_sl])
    pltpu.sync_copy(type_v, type_hbm.at[b, out_sl])


def kernel(tokens1, cu_seqlens1, tokens2, cu_seqlens2, label):
    ids, mask, type_ids = _pack_kernel(tokens1, cu_seqlens1, tokens2, cu_seqlens2)
    return (ids, mask, type_ids, label)


# speedup vs baseline: 5.0632x; 5.0632x over previous
import functools

import jax
import jax.numpy as jnp
from jax import lax
from jax.experimental import pallas as pl
from jax.experimental.pallas import tpu as pltpu
from jax.experimental.pallas import tpu_sc as plsc

SEQ_LEN = 512
CLS_ID = 101
SEP_ID = 102
PAD_ID = 0
B = 16
TOTAL = 4096
BUDGET = SEQ_LEN - 3
HALF = SEQ_LEN // 2
LANES = 16
STEPS = HALF // LANES

_mesh = plsc.VectorSubcoreMesh(core_axis_name="c", subcore_axis_name="s")


@functools.partial(
    pl.kernel,
    mesh=_mesh,
    compiler_params=pltpu.CompilerParams(needs_layout_passes=False),
    out_type=[
        jax.ShapeDtypeStruct((B, SEQ_LEN), jnp.int32),
        jax.ShapeDtypeStruct((B, SEQ_LEN), jnp.int32),
        jax.ShapeDtypeStruct((B, SEQ_LEN), jnp.int32),
    ],
    scratch_types=[
        pltpu.VMEM((128,), jnp.int32),
        pltpu.VMEM((128,), jnp.int32),
        pltpu.VMEM((TOTAL,), jnp.int32),
        pltpu.VMEM((TOTAL,), jnp.int32),
        pltpu.VMEM((HALF,), jnp.int32),
        pltpu.VMEM((HALF,), jnp.int32),
        pltpu.VMEM((HALF,), jnp.int32),
    ],
)
def _pack_kernel(tok1_hbm, cu1_hbm, tok2_hbm, cu2_hbm,
                 ids_hbm, mask_hbm, type_hbm,
                 cu1_v, cu2_v, tok1_v, tok2_v, ids_v, mask_v, type_v):
    wid = lax.axis_index("s") * 2 + lax.axis_index("c")
    b = wid // 2
    h = wid % 2

    pltpu.sync_copy(cu1_hbm, cu1_v.at[pl.ds(0, B + 1)])
    pltpu.sync_copy(cu2_hbm, cu2_v.at[pl.ds(0, B + 1)])
    pltpu.sync_copy(tok1_hbm, tok1_v)
    pltpu.sync_copy(tok2_hbm, tok2_v)

    bv = jnp.full((LANES,), b, jnp.int32)
    c1lo = plsc.load_gather(cu1_v, [bv])
    c1hi = plsc.load_gather(cu1_v, [bv + 1])
    c2lo = plsc.load_gather(cu2_v, [bv])
    c2hi = plsc.load_gather(cu2_v, [bv + 1])
    len1 = c1hi - c1lo
    len2 = c2hi - c2lo
    t1 = jnp.minimum(len1, BUDGET - jnp.minimum(len2, BUDGET // 2))
    t2 = jnp.minimum(len2, BUDGET - t1)
    tsum2 = t1 + t2 + 2

    base = h * HALF
    lane = lax.iota(jnp.int32, LANES)
    for j in range(STEPS):
        p = lane + (j * LANES) + base
        in1 = (p >= 1) & (p <= t1)
        sep1 = p == t1 + 1
        in2 = (p >= t1 + 2) & (p <= tsum2 - 1)
        sep2 = p == tsum2
        idx1 = jnp.clip(c1lo + p - 1, 0, TOTAL - 1)
        idx2 = jnp.clip(c2lo + p - t1 - 2, 0, TOTAL - 1)
        g1 = plsc.load_gather(tok1_v, [idx1])
        g2 = plsc.load_gather(tok2_v, [idx2])
        ids = jnp.where(p == 0, CLS_ID,
              jnp.where(in1, g1,
              jnp.where(sep1, SEP_ID,
              jnp.where(in2, g2,
              jnp.where(sep2, SEP_ID, PAD_ID))))).astype(jnp.int32)
        mask = (p <= tsum2).astype(jnp.int32)
        tids = ((p >= t1 + 2) & (p <= tsum2)).astype(jnp.int32)
        sl = pl.ds(j * LANES, LANES)
        ids_v[sl] = ids
        mask_v[sl] = mask
        type_v[sl] = tids

    out_sl = pl.ds(base, HALF)
    pltpu.sync_copy(ids_v, ids_hbm.at[b, out_sl])
    pltpu.sync_copy(mask_v, mask_hbm.at[b, out_sl])
    pltpu.sync_copy(type_v, type_hbm.at[b, out_sl])


def kernel(tokens1, cu_seqlens1, tokens2, cu_seqlens2, label):
    ids, mask, type_ids = _pack_kernel(tokens1, cu_seqlens1, tokens2, cu_seqlens2)
    return (ids, mask, type_ids, label)


# speedup vs baseline: 5.5509x; 1.0963x over previous
import functools

import jax
import jax.numpy as jnp
from jax import lax
from jax.experimental import pallas as pl
from jax.experimental.pallas import tpu as pltpu
from jax.experimental.pallas import tpu_sc as plsc

SEQ_LEN = 512
CLS_ID = 101
SEP_ID = 102
PAD_ID = 0
B = 16
TOTAL = 4096
BUDGET = SEQ_LEN - 3
HALF = SEQ_LEN // 2
LANES = 16
STEPS = HALF // LANES
WIN = 384

_mesh = plsc.VectorSubcoreMesh(core_axis_name="c", subcore_axis_name="s")


@functools.partial(
    pl.kernel,
    mesh=_mesh,
    compiler_params=pltpu.CompilerParams(needs_layout_passes=False),
    out_type=[
        jax.ShapeDtypeStruct((B, SEQ_LEN), jnp.int32),
        jax.ShapeDtypeStruct((B, SEQ_LEN), jnp.int32),
        jax.ShapeDtypeStruct((B, SEQ_LEN), jnp.int32),
    ],
    scratch_types=[
        pltpu.VMEM((128,), jnp.int32),
        pltpu.VMEM((128,), jnp.int32),
        pltpu.VMEM((WIN,), jnp.int32),
        pltpu.VMEM((WIN,), jnp.int32),
        pltpu.VMEM((HALF,), jnp.int32),
        pltpu.VMEM((HALF,), jnp.int32),
        pltpu.VMEM((HALF,), jnp.int32),
        pltpu.SemaphoreType.DMA,
        pltpu.SemaphoreType.DMA,
        pltpu.SemaphoreType.DMA,
    ],
)
def _pack_kernel(tok1_hbm, cu1_hbm, tok2_hbm, cu2_hbm,
                 ids_hbm, mask_hbm, type_hbm,
                 cu1_v, cu2_v, win1_v, win2_v, ids_v, mask_v, type_v,
                 sem_cu, sem_tok, sem_out):
    wid = lax.axis_index("s") * 2 + lax.axis_index("c")
    b = wid // 2
    h = wid % 2
    base = h * HALF

    hc1 = pltpu.async_copy(cu1_hbm, cu1_v.at[pl.ds(0, B + 1)], sem_cu)
    hc2 = pltpu.async_copy(cu2_hbm, cu2_v.at[pl.ds(0, B + 1)], sem_cu)
    hc1.wait()
    hc2.wait()

    bv = jnp.full((LANES,), b, jnp.int32)
    c1lo = plsc.load_gather(cu1_v, [bv])
    c1hi = plsc.load_gather(cu1_v, [bv + 1])
    c2lo = plsc.load_gather(cu2_v, [bv])
    c2hi = plsc.load_gather(cu2_v, [bv + 1])
    len1 = c1hi - c1lo
    len2 = c2hi - c2lo
    t1 = jnp.minimum(len1, BUDGET - jnp.minimum(len2, BUDGET // 2))
    t2 = jnp.minimum(len2, BUDGET - t1)
    tsum2 = t1 + t2 + 2

    raw1 = jnp.max(c1lo) + base - 1
    raw2 = jnp.max(c2lo - t1) + base - 2
    s1 = pl.multiple_of(jnp.minimum(jnp.maximum(raw1, 0) & ~7, TOTAL - WIN), 8)
    s2 = pl.multiple_of(jnp.minimum(jnp.maximum(raw2, 0) & ~7, TOTAL - WIN), 8)
    hw1 = pltpu.async_copy(tok1_hbm.at[pl.ds(s1, WIN)], win1_v, sem_tok)
    hw2 = pltpu.async_copy(tok2_hbm.at[pl.ds(s2, WIN)], win2_v, sem_tok)

    lane = lax.iota(jnp.int32, LANES)
    for j in range(STEPS):
        p = lane + (j * LANES) + base
        mask = (p <= tsum2).astype(jnp.int32)
        tids = ((p >= t1 + 2) & (p <= tsum2)).astype(jnp.int32)
        sl = pl.ds(j * LANES, LANES)
        mask_v[sl] = mask
        type_v[sl] = tids

    out_sl = pl.ds(base, HALF)
    hm = pltpu.async_copy(mask_v, mask_hbm.at[b, out_sl], sem_out)
    ht = pltpu.async_copy(type_v, type_hbm.at[b, out_sl], sem_out)

    hw1.wait()
    hw2.wait()

    s1v = jnp.full((LANES,), s1, jnp.int32)
    s2v = jnp.full((LANES,), s2, jnp.int32)
    for j in range(STEPS):
        p = lane + (j * LANES) + base
        in1 = (p >= 1) & (p <= t1)
        sep1 = p == t1 + 1
        in2 = (p >= t1 + 2) & (p <= tsum2 - 1)
        sep2 = p == tsum2
        idx1 = jnp.clip(c1lo + p - 1, 0, TOTAL - 1) - s1v
        idx2 = jnp.clip(c2lo + p - t1 - 2, 0, TOTAL - 1) - s2v
        g1 = plsc.load_gather(win1_v, [idx1])
        g2 = plsc.load_gather(win2_v, [idx2])
        ids = jnp.where(p == 0, CLS_ID,
              jnp.where(in1, g1,
              jnp.where(sep1, SEP_ID,
              jnp.where(in2, g2,
              jnp.where(sep2, SEP_ID, PAD_ID))))).astype(jnp.int32)
        ids_v[pl.ds(j * LANES, LANES)] = ids

    hi = pltpu.async_copy(ids_v, ids_hbm.at[b, out_sl], sem_out)
    hm.wait()
    ht.wait()
    hi.wait()


def kernel(tokens1, cu_seqlens1, tokens2, cu_seqlens2, label):
    ids, mask, type_ids = _pack_kernel(tokens1, cu_seqlens1, tokens2, cu_seqlens2)
    return (ids, mask, type_ids, label)
